# baseline (speedup 1.0000x reference)
import functools

import jax
import jax.numpy as jnp
from jax import lax
from jax.experimental import pallas as pl
from jax.experimental.pallas import tpu as pltpu
from jax.experimental.pallas import tpu_sc as plsc

R = 1_000_000
D = 64
U = 65536
NC, NS = 2, 16
NW = NC * NS
CAP = 4096
IDXCH = 4096
CPR = 128
NCHK = 256
CW = CPR * NCHK
CSTRIDE = 31232
WLAST = 967168
TAIL0 = 999936
CWMAX = CW
RING = 7
PB = 64
PBS = 6


def _sc_scatter_copy(memT, idx, val2):
  mesh = plsc.VectorSubcoreMesh(
      core_axis_name="c", subcore_axis_name="s", num_cores=NC,
      num_subcores=NS)

  scratch = [
      pltpu.VMEM((D, RING * CPR), jnp.float32),
      pltpu.VMEM((2 * IDXCH,), jnp.int32),
      pltpu.VMEM((CWMAX,), jnp.int32),
      pltpu.VMEM((CAP,), jnp.int32),
      pltpu.VMEM((CAP,), jnp.int32),
      pltpu.VMEM((CAP,), jnp.int32),
      pltpu.VMEM((2 * PB, 2 * D), jnp.float32),
      pltpu.VMEM((CPR,), jnp.int32),
      pltpu.VMEM((CPR,), jnp.int32),
      pltpu.SemaphoreType.DMA((RING,)),
      pltpu.SemaphoreType.DMA((RING,)),
      pltpu.SemaphoreType.DMA((2,)),
      pltpu.SemaphoreType.DMA((2,)),
  ]

  @functools.partial(
      pl.kernel,
      out_type=jax.ShapeDtypeStruct((D, R), jnp.float32),
      mesh=mesh,
      scratch_types=scratch,
      compiler_params=pltpu.CompilerParams(needs_layout_passes=False),
  )
  def k(memT_hbm, idx_hbm, val2_hbm, outT_hbm, cp_buf, idx_buf, table,
        dest_hits, pos_hits, wpair, pairs, csrc, cdst, rsem, wsem, psem,
        isem):
    core = lax.axis_index("c")
    sid = lax.axis_index("s")
    wid = core * NS + sid
    wlo = jnp.minimum(wid * CSTRIDE, WLAST)
    iot = lax.iota(jnp.int32, 16)
    neg1 = iot * 0 - 1

    def rd(c, d):
      return pltpu.make_async_copy(
          memT_hbm.at[:, pl.ds(wlo + c * CPR, CPR)],
          cp_buf.at[:, pl.ds(d * CPR, CPR)], rsem.at[d])

    def wr(c, d):
      return pltpu.make_async_copy(
          cp_buf.at[:, pl.ds(d * CPR, CPR)],
          outT_hbm.at[:, pl.ds(wlo + c * CPR, CPR)], wsem.at[d])

    def icopy(c, h):
      return pltpu.make_async_copy(
          idx_hbm.at[pl.ds(c * IDXCH, IDXCH)],
          idx_buf.at[pl.ds(h * IDXCH, IDXCH)], isem.at[h])

    def prolog_body(c, carry):
      rd(c, lax.rem(c, RING)).start()
      return carry

    lax.fori_loop(0, RING, prolog_body, jnp.int32(0))
    icopy(0, 0).start()

    def init_body(v, carry):
      table[pl.ds(v * 16, 16)] = neg1
      return carry

    lax.fori_loop(0, CWMAX // 16, init_body, jnp.int32(0))

    cnt = jnp.int32(0)
    for c in range(U // IDXCH):
      h = c % 2
      if c + 1 < U // IDXCH:
        icopy(c + 1, 1 - h).start()
      icopy(c, h).wait()

      def scan_body(v, cnt, c=c, h=h):
        dest = idx_buf[pl.ds(h * IDXCH + v * 16, 16)]
        m = (dest >= wlo) & (dest < wlo + CW)
        nh = jnp.sum(m.astype(jnp.int32))

        @pl.when(nh > 0)
        def _hits():
          pos = c * IDXCH + v * 16 + iot
          pc = plsc.cumsum(m.astype(jnp.int32))
          slots = cnt + pc - 1
          m2 = m & (slots < CAP)
          plsc.store_scatter(dest_hits, [slots], dest - wlo, mask=m2)
          plsc.store_scatter(pos_hits, [slots], pos, mask=m2)

        return jnp.minimum(cnt + nh, CAP)

      cnt = lax.fori_loop(0, IDXCH // 16, scan_body, cnt)

    def dedup_body(v, carry):
      h = v * 16 + iot
      vm = h < cnt
      dest = dest_hits[pl.ds(v * 16, 16)]
      pos = pos_hits[pl.ds(v * 16, 16)]
      dead = h < 0
      for k2 in range(1, 16):
        ahead = jnp.minimum(h + k2, cnt - 1)
        da = plsc.load_gather(dest_hits, [ahead], mask=vm)
        dead = dead | (vm & (h + k2 < cnt) & (da == dest))
      live = vm & ~dead
      plsc.store_scatter(table, [dest], pos, mask=live)
      return carry

    lax.fori_loop(0, (cnt + 15) // 16, dedup_body, jnp.int32(0))

    def stream_body(v, wcnt):
      lp = table[pl.ds(v * 16, 16)]
      m = lp >= 0
      pc = plsc.cumsum(m.astype(jnp.int32))
      sl = wcnt + pc - 1
      m2 = m & (sl < CAP)
      plsc.store_scatter(wpair, [sl], lp >> 1, mask=m2)
      enc = (sl << 1) | (lp & 1)
      plsc.store_scatter(table, [v * 16 + iot],
                         jnp.where(m2, enc, neg1), mask=m)
      return jnp.minimum(wcnt + jnp.sum(m.astype(jnp.int32)), CAP)

    wcnt = lax.fori_loop(0, CWMAX // 16, stream_body, jnp.int32(0))
    nb = (wcnt + PB - 1) // PB

    @pl.when(wcnt > 0)
    def _pad():
      def pad_body(t, carry):
        s = wcnt + t * 16 + iot
        pm = s < nb * PB
        fill = plsc.load_gather(wpair, [jnp.maximum(wcnt - 1, 0) + 0 * iot],
                                mask=pm)
        plsc.store_scatter(wpair, [s], fill, mask=pm)
        return carry

      lax.fori_loop(0, PB // 16, pad_body, jnp.int32(0))

    def pf(b):
      h = lax.rem(b, 2)
      return pltpu.make_async_copy(
          val2_hbm.at[wpair.at[pl.ds(b * PB, PB)]],
          pairs.at[pl.ds(h * PB, PB)], psem.at[h])

    @pl.when(nb > 0)
    def _p0():
      pf(jnp.int32(0)).start()

    @pl.when(nb > 1)
    def _p1():
      pf(jnp.int32(1)).start()

    def collect(base_local, nvec, dcol0):
      nw = jnp.int32(0)
      cmax = jnp.int32(-1)
      for j in range(nvec):
        wix = table[pl.ds(base_local + j * 16, 16)]
        m = wix >= 0
        pc = plsc.cumsum(m.astype(jnp.int32))
        sl = nw + pc - 1
        srcpack = ((wix >> 1) & (2 * PB - 1)) * 256 + (wix & 1) * D
        plsc.store_scatter(csrc, [sl], srcpack, mask=m)
        plsc.store_scatter(cdst, [sl], dcol0 + j * 16 + iot, mask=m)
        cmax = jnp.maximum(cmax, jnp.max(jnp.where(m, wix >> 1, -1)))
        nw = nw + jnp.sum(m.astype(jnp.int32))
      return nw, cmax

    def apply_winners(nw):
      @pl.when(nw > 0)
      def _apply():
        def sweep(s, carry):
          sv = csrc[pl.ds(s * 16, 16)]
          dv = cdst[pl.ds(s * 16, 16)]
          m2 = s * 16 + iot < nw
          rowv = sv >> 8
          colb = sv & 255
          for k2 in range(D):
            vals = plsc.load_gather(pairs, [rowv, colb + k2], mask=m2)
            plsc.store_scatter(cp_buf, [iot * 0 + k2, dv], vals, mask=m2)
          return carry

        lax.fori_loop(0, (nw + 15) // 16, sweep, jnp.int32(0))

    def wait_batches(bw, bn, cmax):
      def wcond(s2):
        bw2, _ = s2
        return bw2 <= (cmax >> PBS)

      def wbody(s2):
        bw2, bn2 = s2
        bn3 = jnp.where(bn2 <= bw2, bn2 + 1, bn2)

        @pl.when(bn2 <= bw2)
        def _():
          pf(bn2).start()

        pf(bw2).wait()
        return (bw2 + 1, bn3)

      return lax.while_loop(wcond, wbody, (bw, bn))

    def ring_body(c, state):
      bw, bn, wmax = state
      d = lax.rem(c, RING)
      rd(c, d).wait()

      nw, cmax = collect(c * CPR, CPR // 16, d * CPR)
      wmax = jnp.maximum(wmax, cmax)
      bw, bn = wait_batches(bw, bn, cmax)
      apply_winners(nw)

      wr(c, d).start()

      def icond(bn2):
        return (bn2 < nb) & (bn2 <= ((wmax - (PB - 1)) >> PBS) + 2)

      def ibody(bn2):
        pf(bn2).start()
        return bn2 + 1

      bn = lax.while_loop(icond, ibody, bn)

      @pl.when((c >= 1) & (c - 1 + RING < NCHK))
      def _refill():
        dp = lax.rem(c - 1, RING)
        wr(c - 1, dp).wait()
        rd(c - 1 + RING, dp).start()

      return (bw, bn, wmax)

    bw, bn, wmax = lax.fori_loop(
        0, NCHK, ring_body, (jnp.int32(0), jnp.int32(2), jnp.int32(-1)))

    def drain_body(e, carry):
      c = NCHK - RING + e
      wr(c, lax.rem(c, RING)).wait()
      return carry

    lax.fori_loop(0, RING, drain_body, jnp.int32(0))

    wait_batches(bw, bn, wcnt - 1)

  return k(memT, idx, val2)


def kernel(mem, idx, val):
  idx = idx.astype(jnp.int32)
  val2 = jnp.reshape(val, (U // 2, 2 * D))
  outT = _sc_scatter_copy(mem.T, idx, val2)
  out = outT.T
  tr = jnp.arange(TAIL0, R, dtype=jnp.int32)
  eq = idx[:, None] == tr[None, :]
  wpos = jnp.max(
      jnp.where(eq, jnp.arange(U, dtype=jnp.int32)[:, None], -1), axis=0)
  tail = jnp.where((wpos >= 0)[:, None], val[jnp.maximum(wpos, 0)],
                   lax.dynamic_slice(mem, (TAIL0, 0), (R - TAIL0, D)))
  return lax.dynamic_update_slice(out, tail, (TAIL0, 0))


# speedup vs baseline: 1.0702x; 1.0702x over previous
import functools

import jax
import jax.numpy as jnp
from jax import lax
from jax.experimental import pallas as pl
from jax.experimental.pallas import tpu as pltpu
from jax.experimental.pallas import tpu_sc as plsc

R = 1_000_000
D = 64
U = 65536
NC, NS = 2, 16
NW = NC * NS
CAP = 4096
IDXCH = 4096
CPR = 128
NCHK = 256
CW = CPR * NCHK
CSTRIDE = 31232
WLAST = 967168
TAIL0 = 999936
CWMAX = CW
RING = 7
PB = 64
PBS = 6


def _sc_scatter_copy(memT, idx, val2):
  mesh = plsc.VectorSubcoreMesh(
      core_axis_name="c", subcore_axis_name="s", num_cores=NC,
      num_subcores=NS)

  scratch = [
      pltpu.VMEM((D, RING * CPR), jnp.float32),
      pltpu.VMEM((2 * IDXCH,), jnp.int32),
      pltpu.VMEM((CWMAX,), jnp.int32),
      pltpu.VMEM((CAP,), jnp.int32),
      pltpu.VMEM((CAP,), jnp.int32),
      pltpu.VMEM((CAP,), jnp.int32),
      pltpu.VMEM((2 * PB, 2 * D), jnp.float32),
      pltpu.VMEM((CPR,), jnp.int32),
      pltpu.VMEM((CPR,), jnp.int32),
      pltpu.SemaphoreType.DMA((RING,)),
      pltpu.SemaphoreType.DMA((RING,)),
      pltpu.SemaphoreType.DMA((2,)),
      pltpu.SemaphoreType.DMA((2,)),
  ]

  @functools.partial(
      pl.kernel,
      out_type=jax.ShapeDtypeStruct((D, R), jnp.float32),
      mesh=mesh,
      scratch_types=scratch,
      compiler_params=pltpu.CompilerParams(needs_layout_passes=False),
  )
  def k(memT_hbm, idx_hbm, val2_hbm, outT_hbm, cp_buf, idx_buf, table,
        dest_hits, pos_hits, wpair, pairs, csrc, cdst, rsem, wsem, psem,
        isem):
    core = lax.axis_index("c")
    sid = lax.axis_index("s")
    wid = core * NS + sid
    wlo = jnp.minimum(wid * CSTRIDE, WLAST)
    iot = lax.iota(jnp.int32, 16)
    neg1 = iot * 0 - 1

    def rd(c, d):
      return pltpu.make_async_copy(
          memT_hbm.at[:, pl.ds(wlo + c * CPR, CPR)],
          cp_buf.at[:, pl.ds(d * CPR, CPR)], rsem.at[d])

    def wr(c, d):
      return pltpu.make_async_copy(
          cp_buf.at[:, pl.ds(d * CPR, CPR)],
          outT_hbm.at[:, pl.ds(wlo + c * CPR, CPR)], wsem.at[d])

    def icopy(c, h):
      return pltpu.make_async_copy(
          idx_hbm.at[pl.ds(c * IDXCH, IDXCH)],
          idx_buf.at[pl.ds(h * IDXCH, IDXCH)], isem.at[h])

    def prolog_body(c, carry):
      rd(c, lax.rem(c, RING)).start()
      return carry

    lax.fori_loop(0, RING, prolog_body, jnp.int32(0))
    icopy(0, 0).start()

    def init_body(v, carry):
      table[pl.ds(v * 16, 16)] = neg1
      return carry

    lax.fori_loop(0, CWMAX // 16, init_body, jnp.int32(0))

    cnt = jnp.int32(0)
    for c in range(U // IDXCH):
      h = c % 2
      if c + 1 < U // IDXCH:
        icopy(c + 1, 1 - h).start()
      icopy(c, h).wait()

      def scan_body(v, cnt, c=c, h=h):
        dest = idx_buf[pl.ds(h * IDXCH + v * 16, 16)]
        pos = c * IDXCH + v * 16 + iot
        m = (dest >= wlo) & (dest < wlo + CW)
        pc = plsc.cumsum(m.astype(jnp.int32))
        slots = cnt + pc - 1
        m = m & (slots < CAP)
        plsc.store_scatter(dest_hits, [slots], dest - wlo, mask=m)
        plsc.store_scatter(pos_hits, [slots], pos, mask=m)
        return jnp.minimum(cnt + jnp.sum(m.astype(jnp.int32)), CAP)

      cnt = lax.fori_loop(0, IDXCH // 16, scan_body, cnt)

    def dedup_body(v, carry):
      h = v * 16 + iot
      vm = h < cnt
      dest = dest_hits[pl.ds(v * 16, 16)]
      pos = pos_hits[pl.ds(v * 16, 16)]
      dead = h < 0
      for k2 in range(1, 16):
        ahead = jnp.minimum(h + k2, cnt - 1)
        da = plsc.load_gather(dest_hits, [ahead], mask=vm)
        dead = dead | (vm & (h + k2 < cnt) & (da == dest))
      live = vm & ~dead
      plsc.store_scatter(table, [dest], pos, mask=live)
      return carry

    lax.fori_loop(0, (cnt + 15) // 16, dedup_body, jnp.int32(0))

    def stream_body(v, wcnt):
      lp = table[pl.ds(v * 16, 16)]
      m = lp >= 0
      pc = plsc.cumsum(m.astype(jnp.int32))
      sl = wcnt + pc - 1
      m2 = m & (sl < CAP)
      plsc.store_scatter(wpair, [sl], lp >> 1, mask=m2)
      enc = (sl << 1) | (lp & 1)
      plsc.store_scatter(table, [v * 16 + iot],
                         jnp.where(m2, enc, neg1), mask=m)
      return jnp.minimum(wcnt + jnp.sum(m.astype(jnp.int32)), CAP)

    wcnt = lax.fori_loop(0, CWMAX // 16, stream_body, jnp.int32(0))
    nb = (wcnt + PB - 1) // PB

    @pl.when(wcnt > 0)
    def _pad():
      def pad_body(t, carry):
        s = wcnt + t * 16 + iot
        pm = s < nb * PB
        fill = plsc.load_gather(wpair, [jnp.maximum(wcnt - 1, 0) + 0 * iot],
                                mask=pm)
        plsc.store_scatter(wpair, [s], fill, mask=pm)
        return carry

      lax.fori_loop(0, PB // 16, pad_body, jnp.int32(0))

    def pf(b):
      h = lax.rem(b, 2)
      return pltpu.make_async_copy(
          val2_hbm.at[wpair.at[pl.ds(b * PB, PB)]],
          pairs.at[pl.ds(h * PB, PB)], psem.at[h])

    @pl.when(nb > 0)
    def _p0():
      pf(jnp.int32(0)).start()

    @pl.when(nb > 1)
    def _p1():
      pf(jnp.int32(1)).start()

    def collect(base_local, nvec, dcol0):
      nw = jnp.int32(0)
      cmax = jnp.int32(-1)
      for j in range(nvec):
        wix = table[pl.ds(base_local + j * 16, 16)]
        m = wix >= 0
        pc = plsc.cumsum(m.astype(jnp.int32))
        sl = nw + pc - 1
        srcpack = ((wix >> 1) & (2 * PB - 1)) * 256 + (wix & 1) * D
        plsc.store_scatter(csrc, [sl], srcpack, mask=m)
        plsc.store_scatter(cdst, [sl], dcol0 + j * 16 + iot, mask=m)
        cmax = jnp.maximum(cmax, jnp.max(jnp.where(m, wix >> 1, -1)))
        nw = nw + jnp.sum(m.astype(jnp.int32))
      return nw, cmax

    def apply_winners(nw):
      @pl.when(nw > 0)
      def _apply():
        def sweep(s, carry):
          sv = csrc[pl.ds(s * 16, 16)]
          dv = cdst[pl.ds(s * 16, 16)]
          m2 = s * 16 + iot < nw
          rowv = sv >> 8
          colb = sv & 255
          for k2 in range(D):
            vals = plsc.load_gather(pairs, [rowv, colb + k2], mask=m2)
            plsc.store_scatter(cp_buf, [iot * 0 + k2, dv], vals, mask=m2)
          return carry

        lax.fori_loop(0, (nw + 15) // 16, sweep, jnp.int32(0))

    def wait_batches(bw, bn, cmax):
      def wcond(s2):
        bw2, _ = s2
        return bw2 <= (cmax >> PBS)

      def wbody(s2):
        bw2, bn2 = s2
        bn3 = jnp.where(bn2 <= bw2, bn2 + 1, bn2)

        @pl.when(bn2 <= bw2)
        def _():
          pf(bn2).start()

        pf(bw2).wait()
        return (bw2 + 1, bn3)

      return lax.while_loop(wcond, wbody, (bw, bn))

    def ring_body(c, state):
      bw, bn, wmax = state
      d = lax.rem(c, RING)
      rd(c, d).wait()

      nw, cmax = collect(c * CPR, CPR // 16, d * CPR)
      wmax = jnp.maximum(wmax, cmax)
      bw, bn = wait_batches(bw, bn, cmax)
      apply_winners(nw)

      wr(c, d).start()

      def icond(bn2):
        return (bn2 < nb) & (bn2 <= ((wmax - (PB - 1)) >> PBS) + 2)

      def ibody(bn2):
        pf(bn2).start()
        return bn2 + 1

      bn = lax.while_loop(icond, ibody, bn)

      @pl.when((c >= 1) & (c - 1 + RING < NCHK))
      def _refill():
        dp = lax.rem(c - 1, RING)
        wr(c - 1, dp).wait()
        rd(c - 1 + RING, dp).start()

      return (bw, bn, wmax)

    bw, bn, wmax = lax.fori_loop(
        0, NCHK, ring_body, (jnp.int32(0), jnp.int32(2), jnp.int32(-1)))

    def drain_body(e, carry):
      c = NCHK - RING + e
      wr(c, lax.rem(c, RING)).wait()
      return carry

    lax.fori_loop(0, RING, drain_body, jnp.int32(0))

    wait_batches(bw, bn, wcnt - 1)

  return k(memT, idx, val2)


def kernel(mem, idx, val):
  idx = idx.astype(jnp.int32)
  val2 = jnp.reshape(val, (U // 2, 2 * D))
  outT = _sc_scatter_copy(mem.T, idx, val2)
  out = outT.T
  tr = jnp.arange(TAIL0, R, dtype=jnp.int32)
  eq = idx[:, None] == tr[None, :]
  wpos = jnp.max(
      jnp.where(eq, jnp.arange(U, dtype=jnp.int32)[:, None], -1), axis=0)
  tail = jnp.where((wpos >= 0)[:, None], val[jnp.maximum(wpos, 0)],
                   lax.dynamic_slice(mem, (TAIL0, 0), (R - TAIL0, D)))
  return lax.dynamic_update_slice(out, tail, (TAIL0, 0))
